# baseline (speedup 1.0000x reference)
import functools

import jax
import jax.numpy as jnp
from jax import lax
from jax.experimental import pallas as pl
from jax.experimental.pallas import tpu as pltpu
from jax.experimental.pallas import tpu_sc as plsc

N = 10000
E = 320000
D = 128
H = 128
G = 64

NC = 2
NS = 16
EPT = E // (NC * NS)
CHUNK = 128
CHUNKS = 80
EPT_PAD = CHUNKS * CHUNK
BUFS = 2
SLAB = 8
ITERS = CHUNKS // (2 * SLAB)
NPAD = N + 16
ROWS_PER_TILE = 624
ROWS_TAIL = N - NS * ROWS_PER_TILE

R = 2000
NBLK = N // R

def _sc_segsum_body(h_hbm, src_hbm, dst_hbm, zeros_hbm, out_hbm,
                    src_v, slab0, slab1, rows_0, rows_1,
                    acc_sh, gsem_0, gsem_1, ssem_0, ssem_1, isem0, isem1):
    rows = (rows_0, rows_1)
    gsem = (gsem_0, gsem_1)
    ssem = (ssem_0, ssem_1)
    cid = lax.axis_index("c")
    sid = lax.axis_index("s")
    pltpu.sync_copy(src_hbm.at[cid, sid], src_v)
    pltpu.async_copy(dst_hbm.at[cid, sid, pl.ds(0, SLAB)], slab0, isem0)
    pltpu.async_copy(dst_hbm.at[cid, sid, pl.ds(SLAB, SLAB)], slab1, isem1)
    rows0 = sid * ROWS_PER_TILE
    tail0 = NS * ROWS_PER_TILE
    pltpu.sync_copy(zeros_hbm.at[pl.ds(rows0, ROWS_PER_TILE)],
                    acc_sh.at[pl.ds(rows0, ROWS_PER_TILE)])

    @pl.when(sid == NS - 1)
    def _():
        pltpu.sync_copy(zeros_hbm.at[pl.ds(tail0, NPAD - tail0)],
                        acc_sh.at[pl.ds(tail0, NPAD - tail0)])

    plsc.subcore_barrier()

    def _src_slice(j):
        return src_v.at[pl.ds(j * CHUNK, CHUNK)]

    for b in range(BUFS):
        pltpu.async_copy(h_hbm.at[_src_slice(b)], rows[b], gsem[b])

    def _do_super(base, slab):
        for r in range(SLAB // BUFS):
            for b in range(BUFS):
                q = r * BUFS + b
                pltpu.make_async_copy(h_hbm.at[_src_slice(base + q)],
                                      rows[b], gsem[b]).wait()
                pltpu.async_copy(rows[b], acc_sh.at[slab.at[q]],
                                 ssem[b], add=True)
            for b in range(BUFS):
                q = r * BUFS + b
                pltpu.make_async_copy(rows[b], acc_sh.at[slab.at[q]],
                                      ssem[b]).wait()

                @pl.when(base + q + BUFS < CHUNKS)
                def _():
                    pltpu.async_copy(h_hbm.at[_src_slice(base + q + BUFS)],
                                     rows[b], gsem[b])

    def body(i, carry):
        base0 = i * 2 * SLAB
        base1 = base0 + SLAB
        pltpu.make_async_copy(dst_hbm.at[cid, sid, pl.ds(base0, SLAB)],
                              slab0, isem0).wait()
        _do_super(base0, slab0)

        @pl.when(i + 1 < ITERS)
        def _():
            pltpu.async_copy(dst_hbm.at[cid, sid,
                                        pl.ds(base0 + 2 * SLAB, SLAB)],
                             slab0, isem0)

        pltpu.make_async_copy(dst_hbm.at[cid, sid, pl.ds(base1, SLAB)],
                              slab1, isem1).wait()
        _do_super(base1, slab1)

        @pl.when(i + 1 < ITERS)
        def _():
            pltpu.async_copy(dst_hbm.at[cid, sid,
                                        pl.ds(base1 + 2 * SLAB, SLAB)],
                             slab1, isem1)

        return carry

    lax.fori_loop(0, ITERS, body, 0)
    plsc.subcore_barrier()
    pltpu.sync_copy(acc_sh.at[pl.ds(rows0, ROWS_PER_TILE)],
                    out_hbm.at[cid, pl.ds(rows0, ROWS_PER_TILE)])

    @pl.when(sid == NS - 1)
    def _():
        pltpu.sync_copy(acc_sh.at[pl.ds(tail0, ROWS_TAIL)],
                        out_hbm.at[cid, pl.ds(tail0, ROWS_TAIL)])


@functools.cache
def _get_sc_segsum():
    mesh = plsc.VectorSubcoreMesh(core_axis_name="c", subcore_axis_name="s")
    return pl.kernel(
        _sc_segsum_body,
        mesh=mesh,
        out_type=jax.ShapeDtypeStruct((NC, N, D), jnp.float32),
        scratch_types=[
            pltpu.VMEM((EPT_PAD,), jnp.int32),
            pltpu.VMEM((SLAB, CHUNK), jnp.int32),
            pltpu.VMEM((SLAB, CHUNK), jnp.int32),
            pltpu.VMEM((CHUNK, D), jnp.float32),
            pltpu.VMEM((CHUNK, D), jnp.float32),
            pltpu.VMEM_SHARED((NPAD, D), jnp.float32),
            pltpu.SemaphoreType.DMA,
            pltpu.SemaphoreType.DMA,
            pltpu.SemaphoreType.DMA,
            pltpu.SemaphoreType.DMA,
            pltpu.SemaphoreType.DMA,
            pltpu.SemaphoreType.DMA,
        ],
    )


def _mlp_body(h_ref, p_ref, w1_ref, b1_ref, w2_ref, b2_ref,
              u_ref, s_ref, ss_ref):
    i = pl.program_id(0)
    z = h_ref[...] + p_ref[0] + p_ref[1]
    a = jnp.maximum(
        lax.dot(z, w1_ref[...], preferred_element_type=jnp.float32)
        + b1_ref[...], 0.0)
    u = jnp.maximum(
        lax.dot(a, w2_ref[...], preferred_element_type=jnp.float32)
        + b2_ref[...], 0.0)
    u_ref[...] = u

    @pl.when(i == 0)
    def _():
        s_ref[...] = jnp.zeros_like(s_ref)
        ss_ref[...] = jnp.zeros_like(ss_ref)

    s_ref[...] += jnp.sum(u, axis=0, keepdims=True)
    ss_ref[...] += jnp.sum(u * u, axis=0, keepdims=True)


_tc_mlp = pl.pallas_call(
    _mlp_body,
    grid=(NBLK,),
    in_specs=[
        pl.BlockSpec((R, D), lambda i: (i, 0)),
        pl.BlockSpec((NC, R, D), lambda i: (0, i, 0)),
        pl.BlockSpec((D, H), lambda i: (0, 0)),
        pl.BlockSpec((1, H), lambda i: (0, 0)),
        pl.BlockSpec((H, H), lambda i: (0, 0)),
        pl.BlockSpec((1, H), lambda i: (0, 0)),
    ],
    out_specs=[
        pl.BlockSpec((R, H), lambda i: (i, 0)),
        pl.BlockSpec((1, H), lambda i: (0, 0)),
        pl.BlockSpec((1, H), lambda i: (0, 0)),
    ],
    out_shape=[
        jax.ShapeDtypeStruct((N, H), jnp.float32),
        jax.ShapeDtypeStruct((1, H), jnp.float32),
        jax.ShapeDtypeStruct((1, H), jnp.float32),
    ],
)


def _bn_scale_shift(s, ss, g, b):
    mean = s / N
    var = ss / N - mean * mean
    scale = g * lax.rsqrt(var + 1e-5)
    shift = b - mean * scale
    return scale, shift


def _bn_body(u_ref, s_ref, ss_ref, g_ref, b_ref, h_ref):
    scale, shift = _bn_scale_shift(s_ref[...], ss_ref[...],
                                   g_ref[...], b_ref[...])
    h_ref[...] = u_ref[...] * scale + shift


_tc_bn = pl.pallas_call(
    _bn_body,
    grid=(NBLK,),
    in_specs=[
        pl.BlockSpec((R, H), lambda i: (i, 0)),
        pl.BlockSpec((1, H), lambda i: (0, 0)),
        pl.BlockSpec((1, H), lambda i: (0, 0)),
        pl.BlockSpec((1, H), lambda i: (0, 0)),
        pl.BlockSpec((1, H), lambda i: (0, 0)),
    ],
    out_specs=pl.BlockSpec((R, H), lambda i: (i, 0)),
    out_shape=jax.ShapeDtypeStruct((N, H), jnp.float32),
)


def _bn_pool_body(u_ref, s_ref, ss_ref, g_ref, b_ref, batch_ref,
                  out_ref, acc, cnt):
    i = pl.program_id(0)

    @pl.when(i == 0)
    def _():
        acc[...] = jnp.zeros_like(acc)
        cnt[...] = jnp.zeros_like(cnt)

    scale, shift = _bn_scale_shift(s_ref[...], ss_ref[...],
                                   g_ref[...], b_ref[...])
    hh = u_ref[...] * scale + shift
    b = batch_ref[0, 0, :]
    m = (b[:, None] == lax.broadcasted_iota(jnp.int32, (R, G), 1)
         ).astype(jnp.float32)
    acc[...] += lax.dot_general(m, hh, (((0,), (0,)), ((), ())),
                                preferred_element_type=jnp.float32)
    cnt[...] += lax.dot_general(m, jnp.ones((R, 1), jnp.float32),
                                (((0,), (0,)), ((), ())),
                                preferred_element_type=jnp.float32)

    @pl.when(i == NBLK - 1)
    def _():
        out_ref[...] = acc[...] / jnp.maximum(cnt[...], 1.0)


_tc_bn_pool = pl.pallas_call(
    _bn_pool_body,
    grid=(NBLK,),
    in_specs=[
        pl.BlockSpec((R, H), lambda i: (i, 0)),
        pl.BlockSpec((1, H), lambda i: (0, 0)),
        pl.BlockSpec((1, H), lambda i: (0, 0)),
        pl.BlockSpec((1, H), lambda i: (0, 0)),
        pl.BlockSpec((1, H), lambda i: (0, 0)),
        pl.BlockSpec((1, 1, R), lambda i: (i, 0, 0)),
    ],
    out_specs=pl.BlockSpec((G, H), lambda i: (0, 0)),
    out_shape=jax.ShapeDtypeStruct((G, H), jnp.float32),
    scratch_shapes=[
        pltpu.VMEM((G, H), jnp.float32),
        pltpu.VMEM((G, 1), jnp.float32),
    ],
)


def kernel(x, edge_index, batch, W1_0, b1_0, W2_0, b2_0, gamma_0, beta_0,
           W1_1, b1_1, W2_1, b2_1, gamma_1, beta_1,
           W1_2, b1_2, W2_2, b2_2, gamma_2, beta_2):
    eidx = edge_index.reshape(2, NC * NS, EPT)
    npad_e = EPT_PAD - EPT
    src_r = jnp.concatenate(
        [eidx[0], jnp.zeros((NC * NS, npad_e), jnp.int32)], axis=1
    ).reshape(NC, NS, EPT_PAD)
    sink = N + (jnp.arange(NC * NS, dtype=jnp.int32) % NS)
    dst_r = jnp.concatenate(
        [eidx[1], jnp.broadcast_to(sink[:, None], (NC * NS, npad_e))], axis=1
    ).reshape(NC, NS, CHUNKS, CHUNK)
    zeros = jnp.zeros((NPAD, D), jnp.float32)
    batch_r = batch.reshape(NBLK, 1, R)
    params = [
        (W1_0, b1_0, W2_0, b2_0, gamma_0, beta_0),
        (W1_1, b1_1, W2_1, b2_1, gamma_1, beta_1),
        (W1_2, b1_2, W2_2, b2_2, gamma_2, beta_2),
    ]
    h = x
    out = None
    for l in range(3):
        W1, b1, W2, b2, gm, bt = params[l]
        p = _get_sc_segsum()(h, src_r, dst_r, zeros)
        u, s, ss = _tc_mlp(h, p, W1, b1.reshape(1, H), W2, b2.reshape(1, H))
        if l < 2:
            h = _tc_bn(u, s, ss, gm.reshape(1, H), bt.reshape(1, H))
        else:
            out = _tc_bn_pool(u, s, ss, gm.reshape(1, H), bt.reshape(1, H),
                              batch_r)
    return out


# speedup vs baseline: 1.0231x; 1.0231x over previous
import functools

import jax
import jax.numpy as jnp
from jax import lax
from jax.experimental import pallas as pl
from jax.experimental.pallas import tpu as pltpu
from jax.experimental.pallas import tpu_sc as plsc

N = 10000
E = 320000
D = 128
H = 128
G = 64

NC = 2
NS = 16
EPT = E // (NC * NS)
CHUNK = 128
CHUNKS = 80
EPT_PAD = CHUNKS * CHUNK
SLAB = 8
NSUPERS = CHUNKS // SLAB
NPAD = N + 16
ROWS_PER_TILE = 624
ROWS_TAIL = N - NS * ROWS_PER_TILE

R = 2000
NBLK = N // R

def _sc_segsum_body(h_hbm, src_hbm, dst_hbm, zeros_hbm, out_hbm,
                    src_v, ring_v, rows_v, acc_sh, gsem, ssem, isem):
    cid = lax.axis_index("c")
    sid = lax.axis_index("s")
    pltpu.sync_copy(src_hbm.at[cid, sid], src_v)
    pltpu.async_copy(dst_hbm.at[cid, sid, pl.ds(0, SLAB)],
                     ring_v.at[pl.ds(0, SLAB)], isem.at[0])
    pltpu.async_copy(dst_hbm.at[cid, sid, pl.ds(SLAB, SLAB)],
                     ring_v.at[pl.ds(SLAB, SLAB)], isem.at[1])
    rows0 = sid * ROWS_PER_TILE
    tail0 = NS * ROWS_PER_TILE
    pltpu.sync_copy(zeros_hbm.at[pl.ds(rows0, ROWS_PER_TILE)],
                    acc_sh.at[pl.ds(rows0, ROWS_PER_TILE)])

    @pl.when(sid == NS - 1)
    def _():
        pltpu.sync_copy(zeros_hbm.at[pl.ds(tail0, NPAD - tail0)],
                        acc_sh.at[pl.ds(tail0, NPAD - tail0)])

    plsc.subcore_barrier()

    def _gather(j, b):
        return pltpu.make_async_copy(
            h_hbm.at[src_v.at[pl.ds(j * CHUNK, CHUNK)]],
            rows_v.at[b], gsem.at[b])

    def _scatter(j, b):
        return pltpu.make_async_copy(
            rows_v.at[b], acc_sh.at[ring_v.at[lax.rem(j, 2 * SLAB)]],
            ssem.at[b])

    _gather(0, 0).start()
    _gather(1, 1).start()
    pltpu.make_async_copy(dst_hbm.at[cid, sid, pl.ds(0, SLAB)],
                          ring_v.at[pl.ds(0, SLAB)], isem.at[0]).wait()
    _gather(0, 0).wait()
    pltpu.async_copy(rows_v.at[0], acc_sh.at[ring_v.at[0]], ssem.at[0],
                     add=True)

    def body(j, carry):
        b = lax.rem(j, 2)
        m = j // SLAB
        @pl.when(lax.rem(j, SLAB) == 0)
        def _():
            pltpu.make_async_copy(
                dst_hbm.at[cid, sid, pl.ds(m * SLAB, SLAB)],
                ring_v.at[pl.ds(lax.rem(m, 2) * SLAB, SLAB)],
                isem.at[lax.rem(m, 2)]).wait()

        _gather(j, b).wait()
        pltpu.async_copy(rows_v.at[b],
                         acc_sh.at[ring_v.at[lax.rem(j, 2 * SLAB)]],
                         ssem.at[b], add=True)
        _scatter(j - 1, 1 - b).wait()

        @pl.when(j + 1 < CHUNKS)
        def _():
            _gather(j + 1, 1 - b).start()

        @pl.when((lax.rem(j, SLAB) == 2) & (m + 1 < NSUPERS))
        def _():
            pltpu.async_copy(
                dst_hbm.at[cid, sid, pl.ds((m + 1) * SLAB, SLAB)],
                ring_v.at[pl.ds(lax.rem(m + 1, 2) * SLAB, SLAB)],
                isem.at[lax.rem(m + 1, 2)])

        return carry

    lax.fori_loop(1, CHUNKS, body, 0)
    _scatter(CHUNKS - 1, lax.rem(CHUNKS - 1, 2)).wait()

    plsc.subcore_barrier()
    pltpu.sync_copy(acc_sh.at[pl.ds(rows0, ROWS_PER_TILE)],
                    out_hbm.at[cid, pl.ds(rows0, ROWS_PER_TILE)])

    @pl.when(sid == NS - 1)
    def _():
        pltpu.sync_copy(acc_sh.at[pl.ds(tail0, ROWS_TAIL)],
                        out_hbm.at[cid, pl.ds(tail0, ROWS_TAIL)])


@functools.cache
def _get_sc_segsum():
    mesh = plsc.VectorSubcoreMesh(core_axis_name="c", subcore_axis_name="s")
    return pl.kernel(
        _sc_segsum_body,
        mesh=mesh,
        out_type=jax.ShapeDtypeStruct((NC, N, D), jnp.float32),
        scratch_types=[
            pltpu.VMEM((EPT_PAD,), jnp.int32),
            pltpu.VMEM((2 * SLAB, CHUNK), jnp.int32),
            pltpu.VMEM((2, CHUNK, D), jnp.float32),
            pltpu.VMEM_SHARED((NPAD, D), jnp.float32),
            pltpu.SemaphoreType.DMA((2,)),
            pltpu.SemaphoreType.DMA((2,)),
            pltpu.SemaphoreType.DMA((2,)),
        ],
    )


def _mlp_body(h_ref, p_ref, w1_ref, b1_ref, w2_ref, b2_ref,
              u_ref, s_ref, ss_ref):
    i = pl.program_id(0)
    z = h_ref[...] + p_ref[0] + p_ref[1]
    a = jnp.maximum(
        lax.dot(z, w1_ref[...], preferred_element_type=jnp.float32)
        + b1_ref[...], 0.0)
    u = jnp.maximum(
        lax.dot(a, w2_ref[...], preferred_element_type=jnp.float32)
        + b2_ref[...], 0.0)
    u_ref[...] = u

    @pl.when(i == 0)
    def _():
        s_ref[...] = jnp.zeros_like(s_ref)
        ss_ref[...] = jnp.zeros_like(ss_ref)

    s_ref[...] += jnp.sum(u, axis=0, keepdims=True)
    ss_ref[...] += jnp.sum(u * u, axis=0, keepdims=True)


_tc_mlp = pl.pallas_call(
    _mlp_body,
    grid=(NBLK,),
    in_specs=[
        pl.BlockSpec((R, D), lambda i: (i, 0)),
        pl.BlockSpec((NC, R, D), lambda i: (0, i, 0)),
        pl.BlockSpec((D, H), lambda i: (0, 0)),
        pl.BlockSpec((1, H), lambda i: (0, 0)),
        pl.BlockSpec((H, H), lambda i: (0, 0)),
        pl.BlockSpec((1, H), lambda i: (0, 0)),
    ],
    out_specs=[
        pl.BlockSpec((R, H), lambda i: (i, 0)),
        pl.BlockSpec((1, H), lambda i: (0, 0)),
        pl.BlockSpec((1, H), lambda i: (0, 0)),
    ],
    out_shape=[
        jax.ShapeDtypeStruct((N, H), jnp.float32),
        jax.ShapeDtypeStruct((1, H), jnp.float32),
        jax.ShapeDtypeStruct((1, H), jnp.float32),
    ],
)


def _bn_scale_shift(s, ss, g, b):
    mean = s / N
    var = ss / N - mean * mean
    scale = g * lax.rsqrt(var + 1e-5)
    shift = b - mean * scale
    return scale, shift


def _bn_body(u_ref, s_ref, ss_ref, g_ref, b_ref, h_ref):
    scale, shift = _bn_scale_shift(s_ref[...], ss_ref[...],
                                   g_ref[...], b_ref[...])
    h_ref[...] = u_ref[...] * scale + shift


_tc_bn = pl.pallas_call(
    _bn_body,
    grid=(NBLK,),
    in_specs=[
        pl.BlockSpec((R, H), lambda i: (i, 0)),
        pl.BlockSpec((1, H), lambda i: (0, 0)),
        pl.BlockSpec((1, H), lambda i: (0, 0)),
        pl.BlockSpec((1, H), lambda i: (0, 0)),
        pl.BlockSpec((1, H), lambda i: (0, 0)),
    ],
    out_specs=pl.BlockSpec((R, H), lambda i: (i, 0)),
    out_shape=jax.ShapeDtypeStruct((N, H), jnp.float32),
)


def _bn_pool_body(u_ref, s_ref, ss_ref, g_ref, b_ref, batch_ref,
                  out_ref, acc, cnt):
    i = pl.program_id(0)

    @pl.when(i == 0)
    def _():
        acc[...] = jnp.zeros_like(acc)
        cnt[...] = jnp.zeros_like(cnt)

    scale, shift = _bn_scale_shift(s_ref[...], ss_ref[...],
                                   g_ref[...], b_ref[...])
    hh = u_ref[...] * scale + shift
    b = batch_ref[0, 0, :]
    m = (b[:, None] == lax.broadcasted_iota(jnp.int32, (R, G), 1)
         ).astype(jnp.float32)
    acc[...] += lax.dot_general(m, hh, (((0,), (0,)), ((), ())),
                                preferred_element_type=jnp.float32)
    cnt[...] += lax.dot_general(m, jnp.ones((R, 1), jnp.float32),
                                (((0,), (0,)), ((), ())),
                                preferred_element_type=jnp.float32)

    @pl.when(i == NBLK - 1)
    def _():
        out_ref[...] = acc[...] / jnp.maximum(cnt[...], 1.0)


_tc_bn_pool = pl.pallas_call(
    _bn_pool_body,
    grid=(NBLK,),
    in_specs=[
        pl.BlockSpec((R, H), lambda i: (i, 0)),
        pl.BlockSpec((1, H), lambda i: (0, 0)),
        pl.BlockSpec((1, H), lambda i: (0, 0)),
        pl.BlockSpec((1, H), lambda i: (0, 0)),
        pl.BlockSpec((1, H), lambda i: (0, 0)),
        pl.BlockSpec((1, 1, R), lambda i: (i, 0, 0)),
    ],
    out_specs=pl.BlockSpec((G, H), lambda i: (0, 0)),
    out_shape=jax.ShapeDtypeStruct((G, H), jnp.float32),
    scratch_shapes=[
        pltpu.VMEM((G, H), jnp.float32),
        pltpu.VMEM((G, 1), jnp.float32),
    ],
)


def kernel(x, edge_index, batch, W1_0, b1_0, W2_0, b2_0, gamma_0, beta_0,
           W1_1, b1_1, W2_1, b2_1, gamma_1, beta_1,
           W1_2, b1_2, W2_2, b2_2, gamma_2, beta_2):
    eidx = edge_index.reshape(2, NC * NS, EPT)
    npad_e = EPT_PAD - EPT
    src_r = jnp.concatenate(
        [eidx[0], jnp.zeros((NC * NS, npad_e), jnp.int32)], axis=1
    ).reshape(NC, NS, EPT_PAD)
    sink = N + (jnp.arange(NC * NS, dtype=jnp.int32) % NS)
    dst_r = jnp.concatenate(
        [eidx[1], jnp.broadcast_to(sink[:, None], (NC * NS, npad_e))], axis=1
    ).reshape(NC, NS, CHUNKS, CHUNK)
    zeros = jnp.zeros((NPAD, D), jnp.float32)
    batch_r = batch.reshape(NBLK, 1, R)
    params = [
        (W1_0, b1_0, W2_0, b2_0, gamma_0, beta_0),
        (W1_1, b1_1, W2_1, b2_1, gamma_1, beta_1),
        (W1_2, b1_2, W2_2, b2_2, gamma_2, beta_2),
    ]
    h = x
    out = None
    for l in range(3):
        W1, b1, W2, b2, gm, bt = params[l]
        p = _get_sc_segsum()(h, src_r, dst_r, zeros)
        u, s, ss = _tc_mlp(h, p, W1, b1.reshape(1, H), W2, b2.reshape(1, H))
        if l < 2:
            h = _tc_bn(u, s, ss, gm.reshape(1, H), bt.reshape(1, H))
        else:
            out = _tc_bn_pool(u, s, ss, gm.reshape(1, H), bt.reshape(1, H),
                              batch_r)
    return out


# speedup vs baseline: 2.5006x; 2.4441x over previous
import functools

import jax
import jax.numpy as jnp
from jax import lax
from jax.experimental import pallas as pl
from jax.experimental.pallas import tpu as pltpu
from jax.experimental.pallas import tpu_sc as plsc

N = 10000
E = 320000
D = 128
H = 128
G = 64

NC = 2
NS = 16
CHUNK = 80
CHUNKS = E // (NC * NS * CHUNK)
EPT = E // (NC * NS)
BUFS = 2
ROUNDS = CHUNKS // BUFS
ROWS_PER_TILE = 624
ROWS_TAIL = N - NS * ROWS_PER_TILE

R = 2000
NBLK = N // R

def _sc_segsum_body(h_hbm, src_hbm, dst_hbm, zeros_hbm, out_hbm,
                    src_v, dst_v, rows_a, rows_b, acc_sh,
                    gsem_a, gsem_b, ssem_a, ssem_b):
    rows = (rows_a, rows_b)
    gsem = (gsem_a, gsem_b)
    ssem = (ssem_a, ssem_b)
    cid = lax.axis_index("c")
    sid = lax.axis_index("s")
    pltpu.sync_copy(src_hbm.at[cid, sid], src_v)
    pltpu.sync_copy(dst_hbm.at[cid, sid], dst_v)
    rows0 = sid * ROWS_PER_TILE
    tail0 = NS * ROWS_PER_TILE
    pltpu.sync_copy(zeros_hbm.at[pl.ds(rows0, ROWS_PER_TILE)],
                    acc_sh.at[pl.ds(rows0, ROWS_PER_TILE)])

    @pl.when(sid == NS - 1)
    def _():
        pltpu.sync_copy(zeros_hbm.at[pl.ds(tail0, ROWS_TAIL)],
                        acc_sh.at[pl.ds(tail0, ROWS_TAIL)])

    plsc.subcore_barrier()

    def _src_slice(j):
        return src_v.at[pl.ds(j * CHUNK, CHUNK)]

    for b in range(BUFS):
        pltpu.async_copy(h_hbm.at[_src_slice(b)], rows[b], gsem[b])

    def body(k, carry):
        j0 = k * BUFS
        for b in range(BUFS):
            pltpu.make_async_copy(h_hbm.at[_src_slice(j0 + b)], rows[b],
                                  gsem[b]).wait()
            pltpu.async_copy(rows[b], acc_sh.at[dst_v.at[j0 + b]],
                             ssem[b], add=True)
        for b in range(BUFS):
            pltpu.make_async_copy(rows[b], acc_sh.at[dst_v.at[j0 + b]],
                                  ssem[b]).wait()

            @pl.when(j0 + BUFS + b < CHUNKS)
            def _():
                pltpu.async_copy(h_hbm.at[_src_slice(j0 + BUFS + b)],
                                 rows[b], gsem[b])

        return carry

    lax.fori_loop(0, ROUNDS, body, 0)
    if CHUNKS % BUFS:
        jlast = ROUNDS * BUFS
        pltpu.make_async_copy(h_hbm.at[_src_slice(jlast)], rows[0],
                              gsem[0]).wait()
        pltpu.sync_copy(rows[0], acc_sh.at[dst_v.at[jlast]], add=True)

    plsc.subcore_barrier()
    pltpu.sync_copy(acc_sh.at[pl.ds(rows0, ROWS_PER_TILE)],
                    out_hbm.at[cid, pl.ds(rows0, ROWS_PER_TILE)])

    @pl.when(sid == NS - 1)
    def _():
        pltpu.sync_copy(acc_sh.at[pl.ds(tail0, ROWS_TAIL)],
                        out_hbm.at[cid, pl.ds(tail0, ROWS_TAIL)])


@functools.cache
def _get_sc_segsum():
    mesh = plsc.VectorSubcoreMesh(core_axis_name="c", subcore_axis_name="s")
    return pl.kernel(
        _sc_segsum_body,
        mesh=mesh,
        out_type=jax.ShapeDtypeStruct((NC, N, D), jnp.float32),
        scratch_types=[
            pltpu.VMEM((EPT,), jnp.int32),
            pltpu.VMEM((CHUNKS, CHUNK), jnp.int32),
            pltpu.VMEM((CHUNK, D), jnp.float32),
            pltpu.VMEM((CHUNK, D), jnp.float32),
            pltpu.VMEM_SHARED((N, D), jnp.float32),
            pltpu.SemaphoreType.DMA,
            pltpu.SemaphoreType.DMA,
            pltpu.SemaphoreType.DMA,
            pltpu.SemaphoreType.DMA,
        ],
    )


def _mlp_body(h_ref, p_ref, w1_ref, b1_ref, w2_ref, b2_ref,
              u_ref, s_ref, ss_ref):
    i = pl.program_id(0)
    z = h_ref[...] + p_ref[0] + p_ref[1]
    a = jnp.maximum(
        lax.dot(z, w1_ref[...], preferred_element_type=jnp.float32)
        + b1_ref[...], 0.0)
    u = jnp.maximum(
        lax.dot(a, w2_ref[...], preferred_element_type=jnp.float32)
        + b2_ref[...], 0.0)
    u_ref[...] = u

    @pl.when(i == 0)
    def _():
        s_ref[...] = jnp.zeros_like(s_ref)
        ss_ref[...] = jnp.zeros_like(ss_ref)

    s_ref[...] += jnp.sum(u, axis=0, keepdims=True)
    ss_ref[...] += jnp.sum(u * u, axis=0, keepdims=True)


_tc_mlp = pl.pallas_call(
    _mlp_body,
    grid=(NBLK,),
    in_specs=[
        pl.BlockSpec((R, D), lambda i: (i, 0)),
        pl.BlockSpec((NC, R, D), lambda i: (0, i, 0)),
        pl.BlockSpec((D, H), lambda i: (0, 0)),
        pl.BlockSpec((1, H), lambda i: (0, 0)),
        pl.BlockSpec((H, H), lambda i: (0, 0)),
        pl.BlockSpec((1, H), lambda i: (0, 0)),
    ],
    out_specs=[
        pl.BlockSpec((R, H), lambda i: (i, 0)),
        pl.BlockSpec((1, H), lambda i: (0, 0)),
        pl.BlockSpec((1, H), lambda i: (0, 0)),
    ],
    out_shape=[
        jax.ShapeDtypeStruct((N, H), jnp.float32),
        jax.ShapeDtypeStruct((1, H), jnp.float32),
        jax.ShapeDtypeStruct((1, H), jnp.float32),
    ],
)


def _bn_scale_shift(s, ss, g, b):
    mean = s / N
    var = ss / N - mean * mean
    scale = g * lax.rsqrt(var + 1e-5)
    shift = b - mean * scale
    return scale, shift


def _bn_body(u_ref, s_ref, ss_ref, g_ref, b_ref, h_ref):
    scale, shift = _bn_scale_shift(s_ref[...], ss_ref[...],
                                   g_ref[...], b_ref[...])
    h_ref[...] = u_ref[...] * scale + shift


_tc_bn = pl.pallas_call(
    _bn_body,
    grid=(NBLK,),
    in_specs=[
        pl.BlockSpec((R, H), lambda i: (i, 0)),
        pl.BlockSpec((1, H), lambda i: (0, 0)),
        pl.BlockSpec((1, H), lambda i: (0, 0)),
        pl.BlockSpec((1, H), lambda i: (0, 0)),
        pl.BlockSpec((1, H), lambda i: (0, 0)),
    ],
    out_specs=pl.BlockSpec((R, H), lambda i: (i, 0)),
    out_shape=jax.ShapeDtypeStruct((N, H), jnp.float32),
)


def _bn_pool_body(u_ref, s_ref, ss_ref, g_ref, b_ref, batch_ref,
                  out_ref, acc, cnt):
    i = pl.program_id(0)

    @pl.when(i == 0)
    def _():
        acc[...] = jnp.zeros_like(acc)
        cnt[...] = jnp.zeros_like(cnt)

    scale, shift = _bn_scale_shift(s_ref[...], ss_ref[...],
                                   g_ref[...], b_ref[...])
    hh = u_ref[...] * scale + shift
    b = batch_ref[0, 0, :]
    m = (b[:, None] == lax.broadcasted_iota(jnp.int32, (R, G), 1)
         ).astype(jnp.float32)
    acc[...] += lax.dot_general(m, hh, (((0,), (0,)), ((), ())),
                                preferred_element_type=jnp.float32)
    cnt[...] += lax.dot_general(m, jnp.ones((R, 1), jnp.float32),
                                (((0,), (0,)), ((), ())),
                                preferred_element_type=jnp.float32)

    @pl.when(i == NBLK - 1)
    def _():
        out_ref[...] = acc[...] / jnp.maximum(cnt[...], 1.0)


_tc_bn_pool = pl.pallas_call(
    _bn_pool_body,
    grid=(NBLK,),
    in_specs=[
        pl.BlockSpec((R, H), lambda i: (i, 0)),
        pl.BlockSpec((1, H), lambda i: (0, 0)),
        pl.BlockSpec((1, H), lambda i: (0, 0)),
        pl.BlockSpec((1, H), lambda i: (0, 0)),
        pl.BlockSpec((1, H), lambda i: (0, 0)),
        pl.BlockSpec((1, 1, R), lambda i: (i, 0, 0)),
    ],
    out_specs=pl.BlockSpec((G, H), lambda i: (0, 0)),
    out_shape=jax.ShapeDtypeStruct((G, H), jnp.float32),
    scratch_shapes=[
        pltpu.VMEM((G, H), jnp.float32),
        pltpu.VMEM((G, 1), jnp.float32),
    ],
)


def kernel(x, edge_index, batch, W1_0, b1_0, W2_0, b2_0, gamma_0, beta_0,
           W1_1, b1_1, W2_1, b2_1, gamma_1, beta_1,
           W1_2, b1_2, W2_2, b2_2, gamma_2, beta_2):
    src_r = edge_index[0].reshape(NC, NS, EPT)
    dst_r = edge_index[1].reshape(NC, NS, CHUNKS, CHUNK)
    zeros = jnp.zeros((N, D), jnp.float32)
    batch_r = batch.reshape(NBLK, 1, R)
    params = [
        (W1_0, b1_0, W2_0, b2_0, gamma_0, beta_0),
        (W1_1, b1_1, W2_1, b2_1, gamma_1, beta_1),
        (W1_2, b1_2, W2_2, b2_2, gamma_2, beta_2),
    ]
    h = x
    out = None
    for l in range(3):
        W1, b1, W2, b2, gm, bt = params[l]
        p = _get_sc_segsum()(h, src_r, dst_r, zeros)
        u, s, ss = _tc_mlp(h, p, W1, b1.reshape(1, H), W2, b2.reshape(1, H))
        if l < 2:
            h = _tc_bn(u, s, ss, gm.reshape(1, H), bt.reshape(1, H))
        else:
            out = _tc_bn_pool(u, s, ss, gm.reshape(1, H), bt.reshape(1, H),
                              batch_r)
    return out


# speedup vs baseline: 2.5044x; 1.0015x over previous
import functools

import jax
import jax.numpy as jnp
from jax import lax
from jax.experimental import pallas as pl
from jax.experimental.pallas import tpu as pltpu
from jax.experimental.pallas import tpu_sc as plsc

N = 10000
E = 320000
D = 128
H = 128
G = 64

NC = 2
NS = 16
CHUNK = 80
CHUNKS = E // (NC * NS * CHUNK)
EPT = E // (NC * NS)
BUFS = 2
ROUNDS = CHUNKS // BUFS
ROWS_PER_TILE = 624
ROWS_TAIL = N - NS * ROWS_PER_TILE

R = 2000
NBLK = N // R

def _sc_segsum_body(h_hbm, src_hbm, dst_hbm, zeros_hbm, out_hbm,
                    src_v, dst_v, rows_a, rows_b, acc_sh,
                    gsem_a, gsem_b, ssem_a, ssem_b):
    rows = (rows_a, rows_b)
    gsem = (gsem_a, gsem_b)
    ssem = (ssem_a, ssem_b)
    cid = lax.axis_index("c")
    sid = lax.axis_index("s")
    pltpu.sync_copy(src_hbm.at[cid, sid], src_v)
    pltpu.sync_copy(dst_hbm.at[cid, sid], dst_v)
    rows0 = sid * ROWS_PER_TILE
    tail0 = NS * ROWS_PER_TILE
    pltpu.sync_copy(zeros_hbm.at[pl.ds(rows0, ROWS_PER_TILE)],
                    acc_sh.at[pl.ds(rows0, ROWS_PER_TILE)])

    @pl.when(sid == NS - 1)
    def _():
        pltpu.sync_copy(zeros_hbm.at[pl.ds(tail0, ROWS_TAIL)],
                        acc_sh.at[pl.ds(tail0, ROWS_TAIL)])

    plsc.subcore_barrier()

    def _src_slice(j):
        return src_v.at[pl.ds(j * CHUNK, CHUNK)]

    for b in range(BUFS):
        pltpu.async_copy(h_hbm.at[_src_slice(b)], rows[b], gsem[b])

    def body(k, carry):
        j0 = k * BUFS
        for b in range(BUFS):
            pltpu.make_async_copy(h_hbm.at[_src_slice(j0 + b)], rows[b],
                                  gsem[b]).wait()
            pltpu.async_copy(rows[b], acc_sh.at[dst_v.at[j0 + b]],
                             ssem[b], add=True)
        for b in range(BUFS):
            pltpu.make_async_copy(rows[b], acc_sh.at[dst_v.at[j0 + b]],
                                  ssem[b]).wait()

            @pl.when(j0 + BUFS + b < CHUNKS)
            def _():
                pltpu.async_copy(h_hbm.at[_src_slice(j0 + BUFS + b)],
                                 rows[b], gsem[b])

        return carry

    lax.fori_loop(0, ROUNDS, body, 0)
    if CHUNKS % BUFS:
        jlast = ROUNDS * BUFS
        pltpu.make_async_copy(h_hbm.at[_src_slice(jlast)], rows[0],
                              gsem[0]).wait()
        pltpu.sync_copy(rows[0], acc_sh.at[dst_v.at[jlast]], add=True)

    plsc.subcore_barrier()
    pltpu.sync_copy(acc_sh.at[pl.ds(rows0, ROWS_PER_TILE)],
                    out_hbm.at[cid, pl.ds(rows0, ROWS_PER_TILE)])

    @pl.when(sid == NS - 1)
    def _():
        pltpu.sync_copy(acc_sh.at[pl.ds(tail0, ROWS_TAIL)],
                        out_hbm.at[cid, pl.ds(tail0, ROWS_TAIL)])


@functools.cache
def _get_sc_segsum():
    mesh = plsc.VectorSubcoreMesh(core_axis_name="c", subcore_axis_name="s")
    return pl.kernel(
        _sc_segsum_body,
        mesh=mesh,
        out_type=jax.ShapeDtypeStruct((NC, N, D), jnp.float32),
        scratch_types=[
            pltpu.VMEM((EPT,), jnp.int32),
            pltpu.VMEM((CHUNKS, CHUNK), jnp.int32),
            pltpu.VMEM((CHUNK, D), jnp.float32),
            pltpu.VMEM((CHUNK, D), jnp.float32),
            pltpu.VMEM_SHARED((N, D), jnp.float32),
            pltpu.SemaphoreType.DMA,
            pltpu.SemaphoreType.DMA,
            pltpu.SemaphoreType.DMA,
            pltpu.SemaphoreType.DMA,
        ],
    )


def _mlp_body(h_ref, p_ref, w1_ref, b1_ref, w2_ref, b2_ref,
              u_ref, s_ref, ss_ref):
    i = pl.program_id(0)
    z = h_ref[...] + p_ref[0] + p_ref[1]
    a = jnp.maximum(
        lax.dot(z, w1_ref[...], preferred_element_type=jnp.float32)
        + b1_ref[...], 0.0)
    u = jnp.maximum(
        lax.dot(a, w2_ref[...], preferred_element_type=jnp.float32)
        + b2_ref[...], 0.0)
    u_ref[...] = u

    @pl.when(i == 0)
    def _():
        s_ref[...] = jnp.zeros_like(s_ref)
        ss_ref[...] = jnp.zeros_like(ss_ref)

    s_ref[...] += jnp.sum(u, axis=0, keepdims=True)
    ss_ref[...] += jnp.sum(u * u, axis=0, keepdims=True)


_tc_mlp = pl.pallas_call(
    _mlp_body,
    grid=(NBLK,),
    in_specs=[
        pl.BlockSpec((R, D), lambda i: (i, 0)),
        pl.BlockSpec((NC, R, D), lambda i: (0, i, 0)),
        pl.BlockSpec((D, H), lambda i: (0, 0)),
        pl.BlockSpec((1, H), lambda i: (0, 0)),
        pl.BlockSpec((H, H), lambda i: (0, 0)),
        pl.BlockSpec((1, H), lambda i: (0, 0)),
    ],
    out_specs=[
        pl.BlockSpec((R, H), lambda i: (i, 0)),
        pl.BlockSpec((1, H), lambda i: (0, 0)),
        pl.BlockSpec((1, H), lambda i: (0, 0)),
    ],
    out_shape=[
        jax.ShapeDtypeStruct((N, H), jnp.float32),
        jax.ShapeDtypeStruct((1, H), jnp.float32),
        jax.ShapeDtypeStruct((1, H), jnp.float32),
    ],
)


def _bn_scale_shift(s, ss, g, b):
    mean = s / N
    var = ss / N - mean * mean
    scale = g / jnp.sqrt(var + 1e-5)
    shift = b - mean * scale
    return scale, shift


def _bn_body(u_ref, s_ref, ss_ref, g_ref, b_ref, h_ref):
    scale, shift = _bn_scale_shift(s_ref[...], ss_ref[...],
                                   g_ref[...], b_ref[...])
    h_ref[...] = u_ref[...] * scale + shift


_tc_bn = pl.pallas_call(
    _bn_body,
    grid=(NBLK,),
    in_specs=[
        pl.BlockSpec((R, H), lambda i: (i, 0)),
        pl.BlockSpec((1, H), lambda i: (0, 0)),
        pl.BlockSpec((1, H), lambda i: (0, 0)),
        pl.BlockSpec((1, H), lambda i: (0, 0)),
        pl.BlockSpec((1, H), lambda i: (0, 0)),
    ],
    out_specs=pl.BlockSpec((R, H), lambda i: (i, 0)),
    out_shape=jax.ShapeDtypeStruct((N, H), jnp.float32),
)


def _bn_pool_body(u_ref, s_ref, ss_ref, g_ref, b_ref, batch_ref,
                  out_ref, acc, cnt):
    i = pl.program_id(0)

    @pl.when(i == 0)
    def _():
        acc[...] = jnp.zeros_like(acc)
        cnt[...] = jnp.zeros_like(cnt)

    scale, shift = _bn_scale_shift(s_ref[...], ss_ref[...],
                                   g_ref[...], b_ref[...])
    hh = u_ref[...] * scale + shift
    b = batch_ref[0, 0, :]
    m = (b[:, None] == lax.broadcasted_iota(jnp.int32, (R, G), 1)
         ).astype(jnp.float32)
    acc[...] += lax.dot_general(m, hh, (((0,), (0,)), ((), ())),
                                preferred_element_type=jnp.float32)
    cnt[...] += lax.dot_general(m, jnp.ones((R, 1), jnp.float32),
                                (((0,), (0,)), ((), ())),
                                preferred_element_type=jnp.float32)

    @pl.when(i == NBLK - 1)
    def _():
        out_ref[...] = acc[...] / jnp.maximum(cnt[...], 1.0)


_tc_bn_pool = pl.pallas_call(
    _bn_pool_body,
    grid=(NBLK,),
    in_specs=[
        pl.BlockSpec((R, H), lambda i: (i, 0)),
        pl.BlockSpec((1, H), lambda i: (0, 0)),
        pl.BlockSpec((1, H), lambda i: (0, 0)),
        pl.BlockSpec((1, H), lambda i: (0, 0)),
        pl.BlockSpec((1, H), lambda i: (0, 0)),
        pl.BlockSpec((1, 1, R), lambda i: (i, 0, 0)),
    ],
    out_specs=pl.BlockSpec((G, H), lambda i: (0, 0)),
    out_shape=jax.ShapeDtypeStruct((G, H), jnp.float32),
    scratch_shapes=[
        pltpu.VMEM((G, H), jnp.float32),
        pltpu.VMEM((G, 1), jnp.float32),
    ],
)


def kernel(x, edge_index, batch, W1_0, b1_0, W2_0, b2_0, gamma_0, beta_0,
           W1_1, b1_1, W2_1, b2_1, gamma_1, beta_1,
           W1_2, b1_2, W2_2, b2_2, gamma_2, beta_2):
    src_r = edge_index[0].reshape(NC, NS, EPT)
    dst_r = edge_index[1].reshape(NC, NS, CHUNKS, CHUNK)
    zeros = jnp.zeros((N, D), jnp.float32)
    batch_r = batch.reshape(NBLK, 1, R)
    params = [
        (W1_0, b1_0, W2_0, b2_0, gamma_0, beta_0),
        (W1_1, b1_1, W2_1, b2_1, gamma_1, beta_1),
        (W1_2, b1_2, W2_2, b2_2, gamma_2, beta_2),
    ]
    h = x
    out = None
    for l in range(3):
        W1, b1, W2, b2, gm, bt = params[l]
        p = _get_sc_segsum()(h, src_r, dst_r, zeros)
        u, s, ss = _tc_mlp(h, p, W1, b1.reshape(1, H), W2, b2.reshape(1, H))
        if l < 2:
            h = _tc_bn(u, s, ss, gm.reshape(1, H), bt.reshape(1, H))
        else:
            out = _tc_bn_pool(u, s, ss, gm.reshape(1, H), bt.reshape(1, H),
                              batch_r)
    return out


# speedup vs baseline: 2.5059x; 1.0006x over previous
import functools

import jax
import jax.numpy as jnp
from jax import lax
from jax.experimental import pallas as pl
from jax.experimental.pallas import tpu as pltpu
from jax.experimental.pallas import tpu_sc as plsc

N = 10000
E = 320000
D = 128
H = 128
G = 64

NC = 2
NS = 16
CHUNK = 80
CHUNKS = E // (NC * NS * CHUNK)
EPT = E // (NC * NS)
BUFS = 2
ROUNDS = CHUNKS // BUFS
ROWS_PER_TILE = 624
ROWS_TAIL = N - NS * ROWS_PER_TILE

R = 2000
NBLK = N // R

def _sc_segsum_body(h_hbm, src_hbm, dst_hbm, zeros_hbm, out_hbm,
                    src_v, dst_v, rows_a, rows_b, acc_sh,
                    gsem_a, gsem_b, ssem_a, ssem_b):
    rows = (rows_a, rows_b)
    gsem = (gsem_a, gsem_b)
    ssem = (ssem_a, ssem_b)
    cid = lax.axis_index("c")
    sid = lax.axis_index("s")
    pltpu.sync_copy(src_hbm.at[cid, sid], src_v)

    def _src_slice(j):
        return src_v.at[pl.ds(j * CHUNK, CHUNK)]

    for b in range(BUFS):
        pltpu.async_copy(h_hbm.at[_src_slice(b)], rows[b], gsem[b])

    pltpu.sync_copy(dst_hbm.at[cid, sid], dst_v)
    rows0 = sid * ROWS_PER_TILE
    tail0 = NS * ROWS_PER_TILE
    pltpu.sync_copy(zeros_hbm.at[pl.ds(rows0, ROWS_PER_TILE)],
                    acc_sh.at[pl.ds(rows0, ROWS_PER_TILE)])

    @pl.when(sid == NS - 1)
    def _():
        pltpu.sync_copy(zeros_hbm.at[pl.ds(tail0, ROWS_TAIL)],
                        acc_sh.at[pl.ds(tail0, ROWS_TAIL)])

    plsc.subcore_barrier()


    def body(k, carry):
        j0 = k * BUFS
        for b in range(BUFS):
            pltpu.make_async_copy(h_hbm.at[_src_slice(j0 + b)], rows[b],
                                  gsem[b]).wait()
            pltpu.async_copy(rows[b], acc_sh.at[dst_v.at[j0 + b]],
                             ssem[b], add=True)
        for b in range(BUFS):
            pltpu.make_async_copy(rows[b], acc_sh.at[dst_v.at[j0 + b]],
                                  ssem[b]).wait()

            @pl.when(j0 + BUFS + b < CHUNKS)
            def _():
                pltpu.async_copy(h_hbm.at[_src_slice(j0 + BUFS + b)],
                                 rows[b], gsem[b])

        return carry

    lax.fori_loop(0, ROUNDS, body, 0)
    if CHUNKS % BUFS:
        jlast = ROUNDS * BUFS
        pltpu.make_async_copy(h_hbm.at[_src_slice(jlast)], rows[0],
                              gsem[0]).wait()
        pltpu.sync_copy(rows[0], acc_sh.at[dst_v.at[jlast]], add=True)

    plsc.subcore_barrier()
    pltpu.sync_copy(acc_sh.at[pl.ds(rows0, ROWS_PER_TILE)],
                    out_hbm.at[cid, pl.ds(rows0, ROWS_PER_TILE)])

    @pl.when(sid == NS - 1)
    def _():
        pltpu.sync_copy(acc_sh.at[pl.ds(tail0, ROWS_TAIL)],
                        out_hbm.at[cid, pl.ds(tail0, ROWS_TAIL)])


@functools.cache
def _get_sc_segsum():
    mesh = plsc.VectorSubcoreMesh(core_axis_name="c", subcore_axis_name="s")
    return pl.kernel(
        _sc_segsum_body,
        mesh=mesh,
        out_type=jax.ShapeDtypeStruct((NC, N, D), jnp.float32),
        scratch_types=[
            pltpu.VMEM((EPT,), jnp.int32),
            pltpu.VMEM((CHUNKS, CHUNK), jnp.int32),
            pltpu.VMEM((CHUNK, D), jnp.float32),
            pltpu.VMEM((CHUNK, D), jnp.float32),
            pltpu.VMEM_SHARED((N, D), jnp.float32),
            pltpu.SemaphoreType.DMA,
            pltpu.SemaphoreType.DMA,
            pltpu.SemaphoreType.DMA,
            pltpu.SemaphoreType.DMA,
        ],
    )


def _mlp_body(h_ref, p_ref, w1_ref, b1_ref, w2_ref, b2_ref,
              u_ref, s_ref, ss_ref):
    i = pl.program_id(0)
    z = h_ref[...] + p_ref[0] + p_ref[1]
    a = jnp.maximum(
        lax.dot(z, w1_ref[...], preferred_element_type=jnp.float32)
        + b1_ref[...], 0.0)
    u = jnp.maximum(
        lax.dot(a, w2_ref[...], preferred_element_type=jnp.float32)
        + b2_ref[...], 0.0)
    u_ref[...] = u

    @pl.when(i == 0)
    def _():
        s_ref[...] = jnp.zeros_like(s_ref)
        ss_ref[...] = jnp.zeros_like(ss_ref)

    s_ref[...] += jnp.sum(u, axis=0, keepdims=True)
    ss_ref[...] += jnp.sum(u * u, axis=0, keepdims=True)


_tc_mlp = pl.pallas_call(
    _mlp_body,
    grid=(NBLK,),
    in_specs=[
        pl.BlockSpec((R, D), lambda i: (i, 0)),
        pl.BlockSpec((NC, R, D), lambda i: (0, i, 0)),
        pl.BlockSpec((D, H), lambda i: (0, 0)),
        pl.BlockSpec((1, H), lambda i: (0, 0)),
        pl.BlockSpec((H, H), lambda i: (0, 0)),
        pl.BlockSpec((1, H), lambda i: (0, 0)),
    ],
    out_specs=[
        pl.BlockSpec((R, H), lambda i: (i, 0)),
        pl.BlockSpec((1, H), lambda i: (0, 0)),
        pl.BlockSpec((1, H), lambda i: (0, 0)),
    ],
    out_shape=[
        jax.ShapeDtypeStruct((N, H), jnp.float32),
        jax.ShapeDtypeStruct((1, H), jnp.float32),
        jax.ShapeDtypeStruct((1, H), jnp.float32),
    ],
)


def _bn_scale_shift(s, ss, g, b):
    mean = s / N
    var = ss / N - mean * mean
    scale = g / jnp.sqrt(var + 1e-5)
    shift = b - mean * scale
    return scale, shift


def _bn_body(u_ref, s_ref, ss_ref, g_ref, b_ref, h_ref):
    scale, shift = _bn_scale_shift(s_ref[...], ss_ref[...],
                                   g_ref[...], b_ref[...])
    h_ref[...] = u_ref[...] * scale + shift


_tc_bn = pl.pallas_call(
    _bn_body,
    grid=(NBLK,),
    in_specs=[
        pl.BlockSpec((R, H), lambda i: (i, 0)),
        pl.BlockSpec((1, H), lambda i: (0, 0)),
        pl.BlockSpec((1, H), lambda i: (0, 0)),
        pl.BlockSpec((1, H), lambda i: (0, 0)),
        pl.BlockSpec((1, H), lambda i: (0, 0)),
    ],
    out_specs=pl.BlockSpec((R, H), lambda i: (i, 0)),
    out_shape=jax.ShapeDtypeStruct((N, H), jnp.float32),
)


def _bn_pool_body(u_ref, s_ref, ss_ref, g_ref, b_ref, batch_ref,
                  out_ref, acc, cnt):
    i = pl.program_id(0)

    @pl.when(i == 0)
    def _():
        acc[...] = jnp.zeros_like(acc)
        cnt[...] = jnp.zeros_like(cnt)

    scale, shift = _bn_scale_shift(s_ref[...], ss_ref[...],
                                   g_ref[...], b_ref[...])
    hh = u_ref[...] * scale + shift
    b = batch_ref[0, 0, :]
    m = (b[:, None] == lax.broadcasted_iota(jnp.int32, (R, G), 1)
         ).astype(jnp.float32)
    acc[...] += lax.dot_general(m, hh, (((0,), (0,)), ((), ())),
                                preferred_element_type=jnp.float32)
    cnt[...] += lax.dot_general(m, jnp.ones((R, 1), jnp.float32),
                                (((0,), (0,)), ((), ())),
                                preferred_element_type=jnp.float32)

    @pl.when(i == NBLK - 1)
    def _():
        out_ref[...] = acc[...] / jnp.maximum(cnt[...], 1.0)


_tc_bn_pool = pl.pallas_call(
    _bn_pool_body,
    grid=(NBLK,),
    in_specs=[
        pl.BlockSpec((R, H), lambda i: (i, 0)),
        pl.BlockSpec((1, H), lambda i: (0, 0)),
        pl.BlockSpec((1, H), lambda i: (0, 0)),
        pl.BlockSpec((1, H), lambda i: (0, 0)),
        pl.BlockSpec((1, H), lambda i: (0, 0)),
        pl.BlockSpec((1, 1, R), lambda i: (i, 0, 0)),
    ],
    out_specs=pl.BlockSpec((G, H), lambda i: (0, 0)),
    out_shape=jax.ShapeDtypeStruct((G, H), jnp.float32),
    scratch_shapes=[
        pltpu.VMEM((G, H), jnp.float32),
        pltpu.VMEM((G, 1), jnp.float32),
    ],
)


def kernel(x, edge_index, batch, W1_0, b1_0, W2_0, b2_0, gamma_0, beta_0,
           W1_1, b1_1, W2_1, b2_1, gamma_1, beta_1,
           W1_2, b1_2, W2_2, b2_2, gamma_2, beta_2):
    src_r = edge_index[0].reshape(NC, NS, EPT)
    dst_r = edge_index[1].reshape(NC, NS, CHUNKS, CHUNK)
    zeros = jnp.zeros((N, D), jnp.float32)
    batch_r = batch.reshape(NBLK, 1, R)
    params = [
        (W1_0, b1_0, W2_0, b2_0, gamma_0, beta_0),
        (W1_1, b1_1, W2_1, b2_1, gamma_1, beta_1),
        (W1_2, b1_2, W2_2, b2_2, gamma_2, beta_2),
    ]
    h = x
    out = None
    for l in range(3):
        W1, b1, W2, b2, gm, bt = params[l]
        p = _get_sc_segsum()(h, src_r, dst_r, zeros)
        u, s, ss = _tc_mlp(h, p, W1, b1.reshape(1, H), W2, b2.reshape(1, H))
        if l < 2:
            h = _tc_bn(u, s, ss, gm.reshape(1, H), bt.reshape(1, H))
        else:
            out = _tc_bn_pool(u, s, ss, gm.reshape(1, H), bt.reshape(1, H),
                              batch_r)
    return out
